# Initial kernel scaffold; baseline (speedup 1.0000x reference)
import jax
import jax.numpy as jnp
from jax.experimental import pallas as pl


def kernel(fluidArea, fluidRestDensity, fluidActualArea, fluidNeighbors, fluidRadialDistances, fluidDistances, dt, support):
    raise NotImplementedError("write your pallas kernel here")


# speedup vs baseline: 57.1651x; 57.1651x over previous
import functools
import math

import jax
import jax.numpy as jnp
from jax import lax
from jax.experimental import pallas as pl
from jax.experimental.pallas import tpu as pltpu
from jax.experimental.pallas import tpu_sc as plsc

NW = 32
CH = 1024
SUB = 128
L = 16


def _ctab_body(area_ref, rho_ref, acta_ref, c_ref):
    a = acta_ref[...]
    c_ref[...] = a * a / (area_ref[...] * rho_ref[...])


def _final_body(sc_ref, area_ref, rho_ref, acta_ref, dt_ref, sup_ref, out_ref):
    x = sc_ref[0, 0] + sc_ref[1, 0]
    y = sc_ref[0, 1] + sc_ref[1, 1]
    s = sc_ref[0, 2] + sc_ref[1, 2]
    dt = dt_ref[0, 0]
    h = sup_ref[0, 0]
    k = -30.0 / (math.pi * h * h * h)
    k2 = k * k
    acta = acta_ref[...]
    fac = -(dt * dt) * acta
    mass = area_ref[...] * rho_ref[...]
    alpha = fac * k2 * ((x * x + y * y) / mass + s)
    out_ref[...] = jnp.clip(alpha, -1.0, -1e-07)


def _sc_edge_kernel(n_pad, e_pad):
    per_w = e_pad // NW
    n_chunks = per_w // CH
    zsl = n_pad // 16

    mesh = plsc.VectorSubcoreMesh(core_axis_name="c", subcore_axis_name="s")

    @functools.partial(
        pl.kernel,
        out_type=jax.ShapeDtypeStruct((6 * n_pad,), jnp.float32),
        mesh=mesh,
        compiler_params=pltpu.CompilerParams(needs_layout_passes=False),
        scratch_types=[
            pltpu.VMEM((n_pad,), jnp.float32),
            pltpu.VMEM((n_pad,), jnp.float32),
            pltpu.VMEM((CH // SUB, SUB), jnp.int32),
            pltpu.VMEM((CH,), jnp.int32),
            pltpu.VMEM((CH,), jnp.float32),
            pltpu.VMEM((CH,), jnp.float32),
            pltpu.VMEM((CH,), jnp.float32),
            pltpu.VMEM((CH,), jnp.float32),
            pltpu.VMEM((CH,), jnp.float32),
            pltpu.VMEM((CH,), jnp.float32),
            pltpu.VMEM((zsl,), jnp.float32),
            pltpu.VMEM_SHARED((n_pad,), jnp.float32),
            pltpu.VMEM_SHARED((n_pad,), jnp.float32),
            pltpu.VMEM_SHARED((n_pad,), jnp.float32),
        ],
    )
    def body(ii2, jj, rr, ddx, ddy, atab_h, ctab_h, out,
             aT, cT, iv, jv, rv, dxv, dyv, sxv, syv, ssv, zb,
             accx, accy, accs):
        c = lax.axis_index("c")
        s = lax.axis_index("s")
        wid = c * 16 + s

        pltpu.sync_copy(atab_h, aT)
        pltpu.sync_copy(ctab_h, cT)

        zero16 = jnp.zeros((L,), jnp.float32)
        for z in range(zsl // L):
            zb[pl.ds(z * L, L)] = zero16
        off = s * zsl
        pltpu.sync_copy(zb, accx.at[pl.ds(off, zsl)])
        pltpu.sync_copy(zb, accy.at[pl.ds(off, zsl)])
        pltpu.sync_copy(zb, accs.at[pl.ds(off, zsl)])
        plsc.subcore_barrier()

        row0 = wid * (per_w // SUB)
        eb0 = wid * per_w

        def chunk(t, carry):
            rb = row0 + t * (CH // SUB)
            eb = eb0 + t * CH
            pltpu.sync_copy(ii2.at[pl.ds(rb, CH // SUB), :], iv)
            pltpu.sync_copy(jj.at[pl.ds(eb, CH)], jv)
            pltpu.sync_copy(rr.at[pl.ds(eb, CH)], rv)
            pltpu.sync_copy(ddx.at[pl.ds(eb, CH)], dxv)
            pltpu.sync_copy(ddy.at[pl.ds(eb, CH)], dyv)
            for k in range(CH // L):
                sl = pl.ds(k * L, L)
                jx = jv[sl]
                a = plsc.load_gather(aT, [jx])
                cc = plsc.load_gather(cT, [jx])
                r16 = rv[sl]
                x16 = dxv[sl]
                y16 = dyv[sl]
                om = 1.0 - r16
                w = om * om
                u = a * w
                sxv[sl] = u * x16
                syv[sl] = u * y16
                ssv[sl] = (cc * (w * w)) * (x16 * x16 + y16 * y16)
            for b in range(CH // SUB):
                bs = pl.ds(b * SUB, SUB)
                idxr = iv.at[b]
                pltpu.sync_copy(sxv.at[bs], accx.at[idxr], add=True)
                pltpu.sync_copy(syv.at[bs], accy.at[idxr], add=True)
                pltpu.sync_copy(ssv.at[bs], accs.at[idxr], add=True)
            return carry

        lax.fori_loop(0, n_chunks, chunk, 0)
        plsc.subcore_barrier()

        sl_acc = pl.ds(off, zsl)
        fb = c * (3 * n_pad) + off
        pltpu.sync_copy(accx.at[sl_acc], zb)
        pltpu.sync_copy(zb, out.at[pl.ds(fb, zsl)])
        pltpu.sync_copy(accy.at[sl_acc], zb)
        pltpu.sync_copy(zb, out.at[pl.ds(fb + n_pad, zsl)])
        pltpu.sync_copy(accs.at[sl_acc], zb)
        pltpu.sync_copy(zb, out.at[pl.ds(fb + 2 * n_pad, zsl)])

    return body


def kernel(fluidArea, fluidRestDensity, fluidActualArea, fluidNeighbors,
           fluidRadialDistances, fluidDistances, dt, support):
    N = fluidArea.shape[0]
    E = fluidNeighbors.shape[1]

    n_pad = ((N + 1 + 255) // 256) * 256
    rows = n_pad // 128
    pad_n = n_pad - N
    area_p = jnp.concatenate([fluidArea, jnp.ones((pad_n,), jnp.float32)])
    rho_p = jnp.concatenate([fluidRestDensity, jnp.ones((pad_n,), jnp.float32)])
    acta_p = jnp.concatenate([fluidActualArea, jnp.zeros((pad_n,), jnp.float32)])

    e_pad = ((E + NW * CH - 1) // (NW * CH)) * (NW * CH)
    pad_e = e_pad - E
    i_e = jnp.concatenate(
        [fluidNeighbors[0], jnp.full((pad_e,), N, jnp.int32)])
    j_e = jnp.concatenate(
        [fluidNeighbors[1], jnp.zeros((pad_e,), jnp.int32)])
    zf = jnp.zeros((pad_e,), jnp.float32)
    r_e = jnp.concatenate([fluidRadialDistances, zf])
    dx_e = jnp.concatenate([fluidDistances[:, 0], zf])
    dy_e = jnp.concatenate([fluidDistances[:, 1], zf])
    ii2 = i_e.reshape(e_pad // SUB, SUB)

    ctab = pl.pallas_call(
        _ctab_body,
        out_shape=jax.ShapeDtypeStruct((rows, 128), jnp.float32),
    )(area_p.reshape(rows, 128), rho_p.reshape(rows, 128),
      acta_p.reshape(rows, 128))

    sc_part = _sc_edge_kernel(n_pad, e_pad)(
        ii2, j_e, r_e, dx_e, dy_e, acta_p, ctab.reshape(n_pad))

    alpha = pl.pallas_call(
        _final_body,
        out_shape=jax.ShapeDtypeStruct((rows, 128), jnp.float32),
        in_specs=[
            pl.BlockSpec(memory_space=pltpu.VMEM),
            pl.BlockSpec(memory_space=pltpu.VMEM),
            pl.BlockSpec(memory_space=pltpu.VMEM),
            pl.BlockSpec(memory_space=pltpu.VMEM),
            pl.BlockSpec(memory_space=pltpu.SMEM),
            pl.BlockSpec(memory_space=pltpu.SMEM),
        ],
        out_specs=pl.BlockSpec(memory_space=pltpu.VMEM),
    )(sc_part.reshape(2, 3, rows, 128), area_p.reshape(rows, 128),
      rho_p.reshape(rows, 128), acta_p.reshape(rows, 128),
      dt.reshape(1, 1), support.reshape(1, 1))

    return alpha.reshape(n_pad)[:N]


# speedup vs baseline: 74.7946x; 1.3084x over previous
import functools
import math

import jax
import jax.numpy as jnp
from jax import lax
from jax.experimental import pallas as pl
from jax.experimental.pallas import tpu as pltpu
from jax.experimental.pallas import tpu_sc as plsc

NW = 32
CH = 1024
SUB = 128
L = 16


def _ctab_body(area_ref, rho_ref, acta_ref, c_ref):
    a = acta_ref[...]
    c_ref[...] = a * a / (area_ref[...] * rho_ref[...])


def _final_body(sc_ref, area_ref, rho_ref, acta_ref, dt_ref, sup_ref, out_ref):
    x = sc_ref[0, 0] + sc_ref[1, 0]
    y = sc_ref[0, 1] + sc_ref[1, 1]
    s = sc_ref[0, 2] + sc_ref[1, 2]
    dt = dt_ref[0, 0]
    h = sup_ref[0, 0]
    k = -30.0 / (math.pi * h * h * h)
    k2 = k * k
    acta = acta_ref[...]
    fac = -(dt * dt) * acta
    mass = area_ref[...] * rho_ref[...]
    alpha = fac * k2 * ((x * x + y * y) / mass + s)
    out_ref[...] = jnp.clip(alpha, -1.0, -1e-07)


def _sc_edge_kernel(n_pad, e_pad):
    per_w = e_pad // NW
    n_chunks = per_w // CH
    assert n_chunks % 2 == 0
    zsl = n_pad // 16

    mesh = plsc.VectorSubcoreMesh(core_axis_name="c", subcore_axis_name="s")

    edge_buf = [
        pltpu.VMEM((CH // SUB, SUB), jnp.int32),
        pltpu.VMEM((CH,), jnp.int32),
        pltpu.VMEM((CH,), jnp.float32),
        pltpu.VMEM((CH,), jnp.float32),
        pltpu.VMEM((CH,), jnp.float32),
        pltpu.VMEM((CH,), jnp.float32),
        pltpu.VMEM((CH,), jnp.float32),
        pltpu.VMEM((CH,), jnp.float32),
    ]

    @functools.partial(
        pl.kernel,
        out_type=jax.ShapeDtypeStruct((6 * n_pad,), jnp.float32),
        mesh=mesh,
        compiler_params=pltpu.CompilerParams(needs_layout_passes=False),
        scratch_types=[
            pltpu.VMEM((n_pad,), jnp.float32),
            pltpu.VMEM((n_pad,), jnp.float32),
        ] + edge_buf + edge_buf + [
            pltpu.VMEM((zsl,), jnp.float32),
            pltpu.VMEM_SHARED((n_pad,), jnp.float32),
            pltpu.VMEM_SHARED((n_pad,), jnp.float32),
            pltpu.VMEM_SHARED((n_pad,), jnp.float32),
            pltpu.SemaphoreType.DMA,
            pltpu.SemaphoreType.DMA,
            pltpu.SemaphoreType.DMA,
        ],
    )
    def body(ii2, jj, rr, ddx, ddy, atab_h, ctab_h, out,
             aT, cT,
             iv0, jv0, rv0, dxv0, dyv0, sxv0, syv0, ssv0,
             iv1, jv1, rv1, dxv1, dyv1, sxv1, syv1, ssv1,
             zb, accx, accy, accs, insem0, insem1, scsem):
        c = lax.axis_index("c")
        s = lax.axis_index("s")
        wid = c * 16 + s

        bufs = [
            (iv0, jv0, rv0, dxv0, dyv0, sxv0, syv0, ssv0, insem0),
            (iv1, jv1, rv1, dxv1, dyv1, sxv1, syv1, ssv1, insem1),
        ]

        row0 = wid * (per_w // SUB)
        eb0 = wid * per_w

        def issue_inputs(t, p):
            iv, jv, rv, dxv, dyv, _, _, _, insem = bufs[p]
            rb = row0 + t * (CH // SUB)
            eb = eb0 + t * CH
            pltpu.async_copy(ii2.at[pl.ds(rb, CH // SUB), :], iv, insem)
            pltpu.async_copy(jj.at[pl.ds(eb, CH)], jv, insem)
            pltpu.async_copy(rr.at[pl.ds(eb, CH)], rv, insem)
            pltpu.async_copy(ddx.at[pl.ds(eb, CH)], dxv, insem)
            pltpu.async_copy(ddy.at[pl.ds(eb, CH)], dyv, insem)

        def wait_inputs(t, p):
            iv, jv, rv, dxv, dyv, _, _, _, insem = bufs[p]
            rb = row0 + t * (CH // SUB)
            eb = eb0 + t * CH
            pltpu.make_async_copy(ii2.at[pl.ds(rb, CH // SUB), :], iv, insem).wait()
            pltpu.make_async_copy(jj.at[pl.ds(eb, CH)], jv, insem).wait()
            pltpu.make_async_copy(rr.at[pl.ds(eb, CH)], rv, insem).wait()
            pltpu.make_async_copy(ddx.at[pl.ds(eb, CH)], dxv, insem).wait()
            pltpu.make_async_copy(ddy.at[pl.ds(eb, CH)], dyv, insem).wait()

        def compute(p):
            iv, jv, rv, dxv, dyv, sxv, syv, ssv, _ = bufs[p]
            for k in range(CH // L):
                sl = pl.ds(k * L, L)
                jx = jv[sl]
                a = plsc.load_gather(aT, [jx])
                cc = plsc.load_gather(cT, [jx])
                r16 = rv[sl]
                x16 = dxv[sl]
                y16 = dyv[sl]
                om = 1.0 - r16
                w = om * om
                u = a * w
                sxv[sl] = u * x16
                syv[sl] = u * y16
                ssv[sl] = (cc * (w * w)) * (x16 * x16 + y16 * y16)

        def scatter(p):
            iv, _, _, _, _, sxv, syv, ssv, _ = bufs[p]
            descs = []
            for b in range(CH // SUB):
                bs = pl.ds(b * SUB, SUB)
                idxr = iv.at[b]
                descs.append(pltpu.async_copy(sxv.at[bs], accx.at[idxr], scsem, add=True))
                descs.append(pltpu.async_copy(syv.at[bs], accy.at[idxr], scsem, add=True))
                descs.append(pltpu.async_copy(ssv.at[bs], accs.at[idxr], scsem, add=True))
            for d in descs:
                d.wait()

        issue_inputs(0, 0)
        pltpu.sync_copy(atab_h, aT)
        pltpu.sync_copy(ctab_h, cT)

        zero16 = jnp.zeros((L,), jnp.float32)
        for z in range(zsl // L):
            zb[pl.ds(z * L, L)] = zero16
        off = s * zsl
        pltpu.sync_copy(zb, accx.at[pl.ds(off, zsl)])
        pltpu.sync_copy(zb, accy.at[pl.ds(off, zsl)])
        pltpu.sync_copy(zb, accs.at[pl.ds(off, zsl)])
        plsc.subcore_barrier()

        n2 = n_chunks // 2

        def chunk2(t2, carry):
            t = 2 * t2
            issue_inputs(t + 1, 1)
            wait_inputs(t, 0)
            compute(0)
            scatter(0)
            @pl.when(t2 < n2 - 1)
            def _():
                issue_inputs(t + 2, 0)
            wait_inputs(t + 1, 1)
            compute(1)
            scatter(1)
            return carry

        lax.fori_loop(0, n2, chunk2, 0)
        plsc.subcore_barrier()

        sl_acc = pl.ds(off, zsl)
        fb = c * (3 * n_pad) + off
        pltpu.sync_copy(accx.at[sl_acc], zb)
        pltpu.sync_copy(zb, out.at[pl.ds(fb, zsl)])
        pltpu.sync_copy(accy.at[sl_acc], zb)
        pltpu.sync_copy(zb, out.at[pl.ds(fb + n_pad, zsl)])
        pltpu.sync_copy(accs.at[sl_acc], zb)
        pltpu.sync_copy(zb, out.at[pl.ds(fb + 2 * n_pad, zsl)])

    return body


def kernel(fluidArea, fluidRestDensity, fluidActualArea, fluidNeighbors,
           fluidRadialDistances, fluidDistances, dt, support):
    N = fluidArea.shape[0]
    E = fluidNeighbors.shape[1]

    n_pad = ((N + 1 + 255) // 256) * 256
    rows = n_pad // 128
    pad_n = n_pad - N
    area_p = jnp.concatenate([fluidArea, jnp.ones((pad_n,), jnp.float32)])
    rho_p = jnp.concatenate([fluidRestDensity, jnp.ones((pad_n,), jnp.float32)])
    acta_p = jnp.concatenate([fluidActualArea, jnp.zeros((pad_n,), jnp.float32)])

    gran = 2 * NW * CH
    e_pad = ((E + gran - 1) // gran) * gran
    pad_e = e_pad - E
    i_e = jnp.concatenate(
        [fluidNeighbors[0], jnp.full((pad_e,), N, jnp.int32)])
    j_e = jnp.concatenate(
        [fluidNeighbors[1], jnp.zeros((pad_e,), jnp.int32)])
    zf = jnp.zeros((pad_e,), jnp.float32)
    r_e = jnp.concatenate([fluidRadialDistances, zf])
    dx_e = jnp.concatenate([fluidDistances[:, 0], zf])
    dy_e = jnp.concatenate([fluidDistances[:, 1], zf])
    ii2 = i_e.reshape(e_pad // SUB, SUB)

    ctab = pl.pallas_call(
        _ctab_body,
        out_shape=jax.ShapeDtypeStruct((rows, 128), jnp.float32),
    )(area_p.reshape(rows, 128), rho_p.reshape(rows, 128),
      acta_p.reshape(rows, 128))

    sc_part = _sc_edge_kernel(n_pad, e_pad)(
        ii2, j_e, r_e, dx_e, dy_e, acta_p, ctab.reshape(n_pad))

    alpha = pl.pallas_call(
        _final_body,
        out_shape=jax.ShapeDtypeStruct((rows, 128), jnp.float32),
        in_specs=[
            pl.BlockSpec(memory_space=pltpu.VMEM),
            pl.BlockSpec(memory_space=pltpu.VMEM),
            pl.BlockSpec(memory_space=pltpu.VMEM),
            pl.BlockSpec(memory_space=pltpu.VMEM),
            pl.BlockSpec(memory_space=pltpu.SMEM),
            pl.BlockSpec(memory_space=pltpu.SMEM),
        ],
        out_specs=pl.BlockSpec(memory_space=pltpu.VMEM),
    )(sc_part.reshape(2, 3, rows, 128), area_p.reshape(rows, 128),
      rho_p.reshape(rows, 128), acta_p.reshape(rows, 128),
      dt.reshape(1, 1), support.reshape(1, 1))

    return alpha.reshape(n_pad)[:N]
